# Initial kernel scaffold; baseline (speedup 1.0000x reference)
import functools

import jax
import jax.numpy as jnp
from jax import lax
from jax.experimental import pallas as pl
from jax.experimental.pallas import tpu as pltpu
from jax.experimental.pallas import tpu_sc as plsc

EMBED = 32
B_TOTAL = 16384 * 50
NUM_WORKERS = 32
B_PER_W = B_TOTAL // NUM_WORKERS
CHUNK = 1600
N_CHUNKS = B_PER_W // CHUNK

_mesh = plsc.VectorSubcoreMesh(core_axis_name="c", subcore_axis_name="s")


@functools.partial(
    pl.kernel,
    out_type=jax.ShapeDtypeStruct((B_TOTAL, EMBED), jnp.float32),
    mesh=_mesh,
    scratch_types=[
        pltpu.VMEM((CHUNK,), jnp.int32),
        pltpu.VMEM((CHUNK, EMBED), jnp.float32),
        pltpu.SemaphoreType.DMA,
    ],
)
def _gather_kernel(x_hbm, w_hbm, out_hbm, idx_v, rows_v, sem):
    wid = lax.axis_index("s") * 2 + lax.axis_index("c")
    base = wid * B_PER_W
    for c in range(N_CHUNKS):
        off = base + c * CHUNK
        pltpu.sync_copy(x_hbm.at[pl.ds(off, CHUNK)], idx_v)
        pltpu.async_copy(w_hbm.at[idx_v], rows_v, sem).wait()
        pltpu.sync_copy(rows_v, out_hbm.at[pl.ds(off, CHUNK)])


def kernel(x, W):
    xf = x.reshape(-1).astype(jnp.int32)
    out = _gather_kernel(xf, W)
    return out.reshape(x.shape[0], x.shape[1], EMBED)


# speedup vs baseline: 1.1022x; 1.1022x over previous
import functools

import jax
import jax.numpy as jnp
from jax import lax
from jax.experimental import pallas as pl
from jax.experimental.pallas import tpu as pltpu
from jax.experimental.pallas import tpu_sc as plsc

EMBED = 32
B_TOTAL = 16384 * 50
NUM_WORKERS = 32
B_PER_W = B_TOTAL // NUM_WORKERS
CHUNK = 1600
N_CHUNKS = B_PER_W // CHUNK

_mesh = plsc.VectorSubcoreMesh(core_axis_name="c", subcore_axis_name="s")


@functools.partial(
    pl.kernel,
    out_type=jax.ShapeDtypeStruct((B_TOTAL, EMBED), jnp.float32),
    mesh=_mesh,
    scratch_types=[
        pltpu.VMEM((CHUNK,), jnp.int32),
        pltpu.VMEM((CHUNK, EMBED), jnp.float32),
        pltpu.SemaphoreType.DMA,
    ],
    compiler_params=pltpu.CompilerParams(use_tc_tiling_on_sc=False),
)
def _gather_kernel(x_hbm, w_hbm, out_hbm, idx_v, rows_v, sem):
    wid = lax.axis_index("s") * 2 + lax.axis_index("c")
    base = wid * B_PER_W
    for c in range(N_CHUNKS):
        off = base + c * CHUNK
        pltpu.sync_copy(x_hbm.at[pl.ds(off, CHUNK)], idx_v)
        pltpu.async_copy(w_hbm.at[idx_v], rows_v, sem).wait()
        pltpu.sync_copy(rows_v, out_hbm.at[pl.ds(off, CHUNK)])


def kernel(x, W):
    xf = x.reshape(-1).astype(jnp.int32)
    out = _gather_kernel(xf, W)
    return out.reshape(x.shape[0], x.shape[1], EMBED)


# speedup vs baseline: 1.7702x; 1.6061x over previous
import functools

import jax
import jax.numpy as jnp
from jax import lax
from jax.experimental import pallas as pl
from jax.experimental.pallas import tpu as pltpu
from jax.experimental.pallas import tpu_sc as plsc

EMBED = 32
SEQ = 50
NROWS = 16384
NUM_WORKERS = 32
ROWS_PER_W = NROWS // NUM_WORKERS
CHUNK_ROWS = 16
N_CHUNKS = ROWS_PER_W // CHUNK_ROWS

_mesh = plsc.VectorSubcoreMesh(core_axis_name="c", subcore_axis_name="s")


@functools.partial(
    pl.kernel,
    out_type=jax.ShapeDtypeStruct((NROWS, SEQ, EMBED), jnp.float32),
    mesh=_mesh,
    scratch_types=[
        pltpu.VMEM((ROWS_PER_W, SEQ), jnp.int32),
        pltpu.VMEM((CHUNK_ROWS, SEQ, EMBED), jnp.float32),
        pltpu.SemaphoreType.DMA,
    ],
    compiler_params=pltpu.CompilerParams(use_tc_tiling_on_sc=False),
)
def _gather_kernel(x_hbm, w_hbm, out_hbm, idx_v, rows_v, sem):
    wid = lax.axis_index("s") * 2 + lax.axis_index("c")
    base = wid * ROWS_PER_W
    pltpu.sync_copy(x_hbm.at[pl.ds(base, ROWS_PER_W)], idx_v)

    def chunk_body(c, _):
        local = c * CHUNK_ROWS
        handles = [
            pltpu.async_copy(
                w_hbm.at[idx_v.at[local + j]], rows_v.at[j], sem
            )
            for j in range(CHUNK_ROWS)
        ]
        for h in handles:
            h.wait()
        pltpu.sync_copy(rows_v, out_hbm.at[pl.ds(base + local, CHUNK_ROWS)])
        return ()

    lax.fori_loop(0, N_CHUNKS, chunk_body, ())


def kernel(x, W):
    return _gather_kernel(x.astype(jnp.int32), W)


# speedup vs baseline: 3.4132x; 1.9281x over previous
import functools

import jax
import jax.numpy as jnp
from jax import lax
from jax.experimental import pallas as pl
from jax.experimental.pallas import tpu as pltpu
from jax.experimental.pallas import tpu_sc as plsc

VOCAB = 1000000
EMBED = 32
SEQ = 50
NROWS = 16384
NUM_CORES = 2
NUM_SUBCORES = 16
E_PER_CORE = EMBED // NUM_CORES
I_PER_TILE = NROWS // NUM_SUBCORES
J_HALF = SEQ // 2
IDX_PER_HALF = J_HALF * I_PER_TILE

_mesh = plsc.VectorSubcoreMesh(core_axis_name="c", subcore_axis_name="s")


@functools.partial(
    pl.kernel,
    out_type=jax.ShapeDtypeStruct((SEQ, EMBED, NROWS), jnp.float32),
    mesh=_mesh,
    scratch_types=[
        pltpu.VMEM((IDX_PER_HALF,), jnp.int32),
        pltpu.VMEM((IDX_PER_HALF,), jnp.float32),
        pltpu.VMEM_SHARED((VOCAB,), jnp.float32),
        pltpu.SemaphoreType.DMA,
    ],
    compiler_params=pltpu.CompilerParams(use_tc_tiling_on_sc=True),
)
def _gather_kernel(xt_hbm, wt_hbm, out_hbm, idx_v, rows_v, w_sh, sem):
    c = lax.axis_index("c")
    s = lax.axis_index("s")
    i0 = s * I_PER_TILE

    for h in range(2):
        j0 = h * J_HALF

        def stage_j(j, _):
            pltpu.sync_copy(
                xt_hbm.at[j0 + j, pl.ds(i0, I_PER_TILE)],
                idx_v.at[pl.ds(j * I_PER_TILE, I_PER_TILE)],
            )
            return ()

        lax.fori_loop(0, J_HALF, stage_j, ())

        def e_body(eo, _):
            e = c * E_PER_CORE + eo
            plsc.subcore_barrier()

            @pl.when(s == 0)
            def _load_row():
                pltpu.sync_copy(wt_hbm.at[e], w_sh)

            plsc.subcore_barrier()
            pltpu.async_copy(w_sh.at[idx_v], rows_v, sem).wait()

            def out_j(j, _):
                pltpu.sync_copy(
                    rows_v.at[pl.ds(j * I_PER_TILE, I_PER_TILE)],
                    out_hbm.at[j0 + j, e, pl.ds(i0, I_PER_TILE)],
                )
                return ()

            lax.fori_loop(0, J_HALF, out_j, ())
            return ()

        lax.fori_loop(0, E_PER_CORE, e_body, ())


def kernel(x, W):
    out_t = _gather_kernel(x.T.astype(jnp.int32), W.T)
    return jnp.transpose(out_t, (2, 0, 1))


# speedup vs baseline: 4.4327x; 1.2987x over previous
import functools

import jax
import jax.numpy as jnp
from jax import lax
from jax.experimental import pallas as pl
from jax.experimental.pallas import tpu as pltpu
from jax.experimental.pallas import tpu_sc as plsc

VOCAB = 1000000
EMBED = 32
SEQ = 50
NROWS = 16384
NUM_CORES = 2
NUM_SUBCORES = 16
E_PER_CORE = EMBED // NUM_CORES
I_PER_TILE = NROWS // NUM_SUBCORES
IDX_PER_TILE = SEQ * I_PER_TILE
J_GROUP = 10
N_GROUPS = SEQ // J_GROUP
G_ELEMS = J_GROUP * I_PER_TILE
_mesh = plsc.VectorSubcoreMesh(core_axis_name="c", subcore_axis_name="s")


@functools.partial(
    pl.kernel,
    out_type=jax.ShapeDtypeStruct((SEQ, EMBED, NROWS), jnp.float32),
    mesh=_mesh,
    scratch_types=[
        pltpu.VMEM((IDX_PER_TILE,), jnp.int32),
        pltpu.VMEM((G_ELEMS,), jnp.float32),
        pltpu.VMEM_SHARED((VOCAB,), jnp.float32),
        pltpu.SemaphoreType.DMA,
        pltpu.SemaphoreType.DMA,
    ],
    compiler_params=pltpu.CompilerParams(use_tc_tiling_on_sc=True),
)
def _gather_kernel(xt_hbm, wt_hbm, out_hbm, idx_v, rows_v, w_sh, sem, sem_o):
    c = lax.axis_index("c")
    s = lax.axis_index("s")
    i0 = s * I_PER_TILE

    def stage_j(j, _):
        pltpu.sync_copy(
            xt_hbm.at[j, pl.ds(i0, I_PER_TILE)],
            idx_v.at[pl.ds(j * I_PER_TILE, I_PER_TILE)],
        )
        return ()

    lax.fori_loop(0, SEQ, stage_j, ())

    def e_body(eo, _):
        e = c * E_PER_CORE + eo
        plsc.subcore_barrier()

        @pl.when(s == 0)
        def _load_row():
            pltpu.sync_copy(wt_hbm.at[e], w_sh)

        plsc.subcore_barrier()

        def group_body(g, _):
            pltpu.async_copy(
                w_sh.at[idx_v.at[pl.ds(g * G_ELEMS, G_ELEMS)]], rows_v, sem
            ).wait()

            def out_j(j, _):
                pltpu.async_copy(
                    rows_v.at[pl.ds(j * I_PER_TILE, I_PER_TILE)],
                    out_hbm.at[g * J_GROUP + j, e, pl.ds(i0, I_PER_TILE)],
                    sem_o,
                )
                return ()

            lax.fori_loop(0, J_GROUP, out_j, ())

            def drain_j(j, _):
                pltpu.make_async_copy(
                    rows_v.at[pl.ds(j * I_PER_TILE, I_PER_TILE)],
                    out_hbm.at[g * J_GROUP + j, e, pl.ds(i0, I_PER_TILE)],
                    sem_o,
                ).wait()
                return ()

            lax.fori_loop(0, J_GROUP, drain_j, ())
            return ()

        lax.fori_loop(0, N_GROUPS, group_body, ())
        return ()

    lax.fori_loop(0, E_PER_CORE, e_body, ())


def kernel(x, W):
    out_t = _gather_kernel(x.T.astype(jnp.int32), W.T)
    return jnp.transpose(out_t, (2, 0, 1))
